# baseline (speedup 1.0000x reference)
import functools

import jax
import jax.numpy as jnp
from jax import lax
from jax.experimental import pallas as pl
from jax.experimental.pallas import tpu as pltpu
from jax.experimental.pallas import tpu_sc as plsc

E = 2048
N = 1024
OUT = 2 * E + N
DT = 1e-06

R = 512
NB = OUT // R
N_KCL = N // R
N_KVL = E // R
N_EL = E // R

_SC_INFO = plsc.get_sparse_core_info()
_NC = _SC_INFO.num_cores
_NS = _SC_INFO.num_subcores
_NW = _NC * _NS
_CHUNK = E // _NW
_LANES = 16


def _zy_sc_kernel(p_hbm, k_hbm, s_hbm, z_hbm, y_hbm, p_v, k_v, s_v, z_v, y_v):
    wid = lax.axis_index("s") * _NC + lax.axis_index("c")
    base = wid * _CHUNK
    pltpu.sync_copy(p_hbm.at[pl.ds(base, _CHUNK)], p_v)
    pltpu.sync_copy(k_hbm.at[pl.ds(base, _CHUNK)], k_v)
    pltpu.sync_copy(s_hbm.at[pl.ds(base, _CHUNK)], s_v)
    for j in range(_CHUNK // _LANES):
        sl = pl.ds(j * _LANES, _LANES)
        params = p_v[sl]
        kinds = k_v[sl]
        sw_on = s_v[sl] > 0.0
        sw_off_f = jnp.where(sw_on, 0.0, 1.0)
        sw_on_f = jnp.where(sw_on, 1.0, 0.0)
        z_v[sl] = jnp.where(kinds == 0, -params,
                  jnp.where(kinds == 4, -DT / params,
                  jnp.where(kinds == 5, 1.0,
                  jnp.where(kinds == 2, 1.0,
                  jnp.where(kinds == 3, sw_off_f, 0.0)))))
        y_v[sl] = jnp.where(kinds == 0, 1.0,
                  jnp.where(kinds == 4, 1.0,
                  jnp.where(kinds == 5, -DT / params,
                  jnp.where(kinds == 1, 1.0,
                  jnp.where(kinds == 3, sw_on_f, 0.0)))))
    pltpu.sync_copy(z_v, z_hbm.at[pl.ds(base, _CHUNK)])
    pltpu.sync_copy(y_v, y_hbm.at[pl.ds(base, _CHUNK)])


def _compute_zy(params, kinds, swcol):
    return pl.kernel(
        _zy_sc_kernel,
        out_type=(jax.ShapeDtypeStruct((E,), jnp.float32),
                  jax.ShapeDtypeStruct((E,), jnp.float32)),
        mesh=plsc.VectorSubcoreMesh(core_axis_name="c", subcore_axis_name="s"),
        scratch_types=[
            pltpu.VMEM((_CHUNK,), jnp.float32),
            pltpu.VMEM((_CHUNK,), jnp.int32),
            pltpu.VMEM((_CHUNK,), jnp.float32),
            pltpu.VMEM((_CHUNK,), jnp.float32),
            pltpu.VMEM((_CHUNK,), jnp.float32),
        ],
    )(params, kinds, swcol)


def _band_kernel(m_hbm, z_ref, y_ref, out_ref, m_vmem, sem):
    s = pl.program_id(0)

    @pl.when(s == 0)
    def _start_m_copy():
        pltpu.make_async_copy(m_hbm, m_vmem, sem).start()

    @pl.when(s == N_EL)
    def _wait_m_copy():
        pltpu.make_async_copy(m_hbm, m_vmem, sem).wait()

    @pl.when(s < N_EL)
    def _el():
        e0 = s * R
        rows = jax.lax.broadcasted_iota(jnp.int32, (R, E), 0)
        cols = jax.lax.broadcasted_iota(jnp.int32, (R, E), 1)
        diag = cols == rows + e0
        out_ref[:, 0:E] = jnp.where(diag, z_ref[...], 0.0)
        out_ref[:, E:2 * E] = jnp.where(diag, y_ref[...], 0.0)
        out_ref[:, 2 * E:] = jnp.zeros((R, N), jnp.float32)

    def _kvl(e0):
        rows = jax.lax.broadcasted_iota(jnp.int32, (R, E), 0)
        cols = jax.lax.broadcasted_iota(jnp.int32, (R, E), 1)
        out_ref[:, 0:E] = jnp.zeros((R, E), jnp.float32)
        out_ref[:, E:2 * E] = jnp.where(cols == rows + e0, 1.0, 0.0)
        out_ref[:, 2 * E:] = -m_vmem[:, e0:e0 + R].T

    def _kcl(r0):
        out_ref[:, 0:E] = m_vmem[r0:r0 + R, :]
        out_ref[:, E:] = jnp.zeros((R, OUT - E), jnp.float32)

    for b in range(N_KVL):
        pl.when(s == N_EL + b)(lambda b=b: _kvl(b * R))
    for b in range(N_KCL):
        pl.when(s == N_EL + N_KVL + b)(lambda b=b: _kcl(b * R))


def _out_band(s):
    return jnp.where(s < N_EL, s + N_KCL + N_KVL,
           jnp.where(s < N_EL + N_KVL, s - N_EL + N_KCL,
                     s - N_EL - N_KVL))


def kernel(M, params, sw_params, kinds, time):
    swcol = sw_params[:, time]
    z, y = _compute_zy(params.astype(jnp.float32),
                       kinds.astype(jnp.int32),
                       swcol.astype(jnp.float32))
    z2 = z.reshape(1, E)
    y2 = y.reshape(1, E)

    out = pl.pallas_call(
        _band_kernel,
        grid=(NB,),
        in_specs=[
            pl.BlockSpec(memory_space=pl.ANY),
            pl.BlockSpec((1, E), lambda i: (0, 0)),
            pl.BlockSpec((1, E), lambda i: (0, 0)),
        ],
        out_specs=pl.BlockSpec((R, OUT), lambda i: (_out_band(i), 0)),
        out_shape=jax.ShapeDtypeStruct((OUT, OUT), jnp.float32),
        scratch_shapes=[
            pltpu.VMEM((N, E), jnp.float32),
            pltpu.SemaphoreType.DMA,
        ],
    )(M, z2, y2)
    return out


# speedup vs baseline: 1.4831x; 1.4831x over previous
import jax
import jax.numpy as jnp
from jax.experimental import pallas as pl
from jax.experimental.pallas import tpu as pltpu

E = 2048
N = 1024
OUT = 2 * E + N
DT = 1e-06

R = 512
C = OUT // 2
NB = OUT // R
N_KCL = N // R
N_KVL = E // R
N_EL = E // R


def _band_kernel(m_hbm, p_ref, k_ref, s_ref, out_ref, m_vmem, zy_vmem, sem):
    i = pl.program_id(0)
    j = pl.program_id(1)

    @pl.when(jnp.logical_and(i == 0, j == 0))
    def _first_step():
        pltpu.make_async_copy(m_hbm, m_vmem, sem).start()
        params = p_ref[...]
        kinds = k_ref[...]
        sw_on = s_ref[...] > 0.0
        z = jnp.where(kinds == 0, -params,
            jnp.where(kinds == 4, -DT / params,
            jnp.where(kinds == 5, 1.0,
            jnp.where(kinds == 2, 1.0,
            jnp.where(jnp.logical_and(kinds == 3, jnp.logical_not(sw_on)),
                      1.0, 0.0)))))
        y = jnp.where(kinds == 0, 1.0,
            jnp.where(kinds == 4, 1.0,
            jnp.where(kinds == 5, -DT / params,
            jnp.where(kinds == 1, 1.0,
            jnp.where(jnp.logical_and(kinds == 3, sw_on), 1.0, 0.0)))))
        zy_vmem[:, 0:E] = z
        zy_vmem[:, E:2 * E] = y
        zy_vmem[:, 2 * E:] = jnp.zeros((1, N), jnp.float32)

    @pl.when(jnp.logical_and(i == N_EL, j == 1))
    def _wait_m_copy():
        pltpu.make_async_copy(m_hbm, m_vmem, sem).wait()

    rows = jax.lax.broadcasted_iota(jnp.int32, (R, C), 0)
    cols = jax.lax.broadcasted_iota(jnp.int32, (R, C), 1)

    def _el(e0, c0):
        gc = cols + c0
        diag = jnp.logical_or(gc == rows + e0, gc == rows + (E + e0))
        out_ref[...] = jnp.where(diag, zy_vmem[0:1, c0:c0 + C], 0.0)

    def _kvl(e0, c0):
        gc = cols + c0
        out_ref[...] = jnp.where(gc == rows + (E + e0), 1.0, 0.0)
        if c0 + C > 2 * E:
            out_ref[:, 2 * E - c0:] = -m_vmem[:, e0:e0 + R].T

    def _kcl(r0, c0):
        if c0 == 0:
            out_ref[:, 0:E] = m_vmem[r0:r0 + R, :]
            out_ref[:, E:] = jnp.zeros((R, C - E), jnp.float32)
        else:
            out_ref[...] = jnp.zeros((R, C), jnp.float32)

    for b in range(N_EL):
        for jj in range(2):
            pl.when(jnp.logical_and(i == b, j == jj))(
                lambda b=b, jj=jj: _el(b * R, jj * C))
    for b in range(N_KVL):
        for jj in range(2):
            pl.when(jnp.logical_and(i == N_EL + b, j == jj))(
                lambda b=b, jj=jj: _kvl(b * R, jj * C))
    for b in range(N_KCL):
        for jj in range(2):
            pl.when(jnp.logical_and(i == N_EL + N_KVL + b, j == jj))(
                lambda b=b, jj=jj: _kcl(b * R, jj * C))


def _out_band(s):
    return jnp.where(s < N_EL, s + N_KCL + N_KVL,
           jnp.where(s < N_EL + N_KVL, s - N_EL + N_KCL,
                     s - N_EL - N_KVL))


def kernel(M, params, sw_params, kinds, time):
    swcol = sw_params[:, time]
    p2 = params.reshape(1, E).astype(jnp.float32)
    k2 = kinds.reshape(1, E).astype(jnp.int32)
    s2 = swcol.reshape(1, E).astype(jnp.float32)

    out = pl.pallas_call(
        _band_kernel,
        grid=(NB, 2),
        in_specs=[
            pl.BlockSpec(memory_space=pl.ANY),
            pl.BlockSpec((1, E), lambda i, j: (0, 0)),
            pl.BlockSpec((1, E), lambda i, j: (0, 0)),
            pl.BlockSpec((1, E), lambda i, j: (0, 0)),
        ],
        out_specs=pl.BlockSpec((R, C), lambda i, j: (_out_band(i), j)),
        out_shape=jax.ShapeDtypeStruct((OUT, OUT), jnp.float32),
        scratch_shapes=[
            pltpu.VMEM((N, E), jnp.float32),
            pltpu.VMEM((1, OUT), jnp.float32),
            pltpu.SemaphoreType.DMA,
        ],
    )(M, p2, k2, s2)
    return out


# speedup vs baseline: 1.4929x; 1.0066x over previous
import jax
import jax.numpy as jnp
from jax.experimental import pallas as pl
from jax.experimental.pallas import tpu as pltpu

E = 2048
N = 1024
OUT = 2 * E + N
DT = 1e-06

R = 512
C = OUT // 2
NB = OUT // R
N_KCL = N // R
N_KVL = E // R
N_EL = E // R


def _diag_hits_tile(d0, c0):
    return d0 + R > c0 and d0 < c0 + C


def _band_kernel(m_hbm, p_ref, k_ref, s_ref, out_ref, m_vmem, zy_vmem, sem):
    i = pl.program_id(0)
    j = pl.program_id(1)

    @pl.when(jnp.logical_and(i == 0, j == 0))
    def _first_step():
        pltpu.make_async_copy(m_hbm, m_vmem, sem).start()
        params = p_ref[...]
        kinds = k_ref[...]
        sw_on = s_ref[...] > 0.0
        z = jnp.where(kinds == 0, -params,
            jnp.where(kinds == 4, -DT / params,
            jnp.where(kinds == 5, 1.0,
            jnp.where(kinds == 2, 1.0,
            jnp.where(jnp.logical_and(kinds == 3, jnp.logical_not(sw_on)),
                      1.0, 0.0)))))
        y = jnp.where(kinds == 0, 1.0,
            jnp.where(kinds == 4, 1.0,
            jnp.where(kinds == 5, -DT / params,
            jnp.where(kinds == 1, 1.0,
            jnp.where(jnp.logical_and(kinds == 3, sw_on), 1.0, 0.0)))))
        zy_vmem[:, 0:E] = z
        zy_vmem[:, E:2 * E] = y
        zy_vmem[:, 2 * E:] = jnp.zeros((1, N), jnp.float32)

    @pl.when(jnp.logical_and(i == N_EL, j == 1))
    def _wait_m_copy():
        pltpu.make_async_copy(m_hbm, m_vmem, sem).wait()

    def _diag_mask(offsets):
        rows = jax.lax.broadcasted_iota(jnp.int32, (R, C), 0)
        cols = jax.lax.broadcasted_iota(jnp.int32, (R, C), 1)
        m = cols == rows + offsets[0]
        for d in offsets[1:]:
            m = jnp.logical_or(m, cols == rows + d)
        return m

    def _el(e0, c0):
        offs = [d - c0 for d in (e0, E + e0) if _diag_hits_tile(d, c0)]
        if offs:
            out_ref[...] = jnp.where(_diag_mask(offs),
                                     zy_vmem[0:1, c0:c0 + C], 0.0)
        else:
            out_ref[...] = jnp.zeros((R, C), jnp.float32)

    def _kvl(e0, c0):
        if _diag_hits_tile(E + e0, c0):
            out_ref[...] = jnp.where(_diag_mask([E + e0 - c0]), 1.0, 0.0)
        else:
            out_ref[...] = jnp.zeros((R, C), jnp.float32)
        if c0 + C > 2 * E:
            out_ref[:, 2 * E - c0:] = -m_vmem[:, e0:e0 + R].T

    def _kcl(r0, c0):
        if c0 == 0:
            out_ref[:, 0:E] = m_vmem[r0:r0 + R, :]
            out_ref[:, E:] = jnp.zeros((R, C - E), jnp.float32)
        else:
            out_ref[...] = jnp.zeros((R, C), jnp.float32)

    for b in range(N_EL):
        for jj in range(2):
            pl.when(jnp.logical_and(i == b, j == jj))(
                lambda b=b, jj=jj: _el(b * R, jj * C))
    for b in range(N_KVL):
        for jj in range(2):
            pl.when(jnp.logical_and(i == N_EL + b, j == jj))(
                lambda b=b, jj=jj: _kvl(b * R, jj * C))
    for b in range(N_KCL):
        for jj in range(2):
            pl.when(jnp.logical_and(i == N_EL + N_KVL + b, j == jj))(
                lambda b=b, jj=jj: _kcl(b * R, jj * C))


def _out_band(s):
    return jnp.where(s < N_EL, s + N_KCL + N_KVL,
           jnp.where(s < N_EL + N_KVL, s - N_EL + N_KCL,
                     s - N_EL - N_KVL))


def kernel(M, params, sw_params, kinds, time):
    swcol = sw_params[:, time]
    p2 = params.reshape(1, E).astype(jnp.float32)
    k2 = kinds.reshape(1, E).astype(jnp.int32)
    s2 = swcol.reshape(1, E).astype(jnp.float32)

    out = pl.pallas_call(
        _band_kernel,
        grid=(NB, 2),
        in_specs=[
            pl.BlockSpec(memory_space=pl.ANY),
            pl.BlockSpec((1, E), lambda i, j: (0, 0)),
            pl.BlockSpec((1, E), lambda i, j: (0, 0)),
            pl.BlockSpec((1, E), lambda i, j: (0, 0)),
        ],
        out_specs=pl.BlockSpec((R, C), lambda i, j: (_out_band(i), j)),
        out_shape=jax.ShapeDtypeStruct((OUT, OUT), jnp.float32),
        scratch_shapes=[
            pltpu.VMEM((N, E), jnp.float32),
            pltpu.VMEM((1, OUT), jnp.float32),
            pltpu.SemaphoreType.DMA,
        ],
    )(M, p2, k2, s2)
    return out
